# baseline (speedup 1.0000x reference)
import functools

import jax
import jax.numpy as jnp
from jax import lax
from jax.experimental import pallas as pl
from jax.experimental.pallas import tpu as pltpu
from jax.experimental.pallas import tpu_sc as plsc

_NUM_CORES = 2
_NUM_SUBCORES = 16
_NUM_WORKERS = _NUM_CORES * _NUM_SUBCORES


@functools.lru_cache(maxsize=None)
def _make_sc_gather(V, D, B, chunk, group):
  assert B % _NUM_WORKERS == 0
  b_per_w = B // _NUM_WORKERS
  cpg = group // chunk
  n_groups = b_per_w // group
  assert b_per_w % group == 0 and group % chunk == 0

  mesh = plsc.VectorSubcoreMesh(
      core_axis_name="c", subcore_axis_name="s",
      num_cores=_NUM_CORES, num_subcores=_NUM_SUBCORES)

  @functools.partial(
      pl.kernel,
      mesh=mesh,
      out_type=jax.ShapeDtypeStruct((B, D), jnp.float32),
      scratch_types=[
          pltpu.VMEM((b_per_w,), jnp.int32),
          pltpu.VMEM((2, group, D), jnp.float32),
          pltpu.SemaphoreType.DMA,
          pltpu.SemaphoreType.DMA,
      ],
  )
  def gather_kernel(table_hbm, idx_hbm, out_hbm, idx_v, rows_v, gsem, wsem):
    wid = lax.axis_index("s") * _NUM_CORES + lax.axis_index("c")
    base = wid * b_per_w
    pltpu.sync_copy(idx_hbm.at[pl.ds(base, b_per_w)], idx_v)

    def start_group(g):
      buf = rows_v.at[g % 2]
      return [
          pltpu.async_copy(
              table_hbm.at[idx_v.at[pl.ds(g * group + c * chunk, chunk)]],
              buf.at[pl.ds(c * chunk, chunk)], gsem)
          for c in range(cpg)
      ]

    gathers = start_group(0)
    writes = [None] * n_groups
    for g in range(n_groups):
      if g + 1 < n_groups:
        if g >= 1:
          writes[g - 1].wait()
        nxt = start_group(g + 1)
      for cp in gathers:
        cp.wait()
      writes[g] = pltpu.async_copy(
          rows_v.at[g % 2], out_hbm.at[pl.ds(base + g * group, group)], wsem)
      if g + 1 < n_groups:
        gathers = nxt
    writes[n_groups - 2].wait()
    writes[n_groups - 1].wait()

  return gather_kernel


def kernel(word_inputs, word_seq_lengths, char_inputs, char_seq_lengths,
           char_seq_recover, W):
  B, S = word_inputs.shape
  V, D = W.shape
  s_pad = (S + 7) // 8 * 8
  idx = jnp.pad(word_inputs.astype(jnp.int32),
                ((0, 0), (0, s_pad - S))).reshape(B * s_pad)
  out = _make_sc_gather(V, D, B * s_pad, 112, 224)(W, idx)
  return out.reshape(B, s_pad, D)[:, :S, :]


# speedup vs baseline: 4.9213x; 4.9213x over previous
import functools

import jax
import jax.numpy as jnp
from jax import lax
from jax.experimental import pallas as pl
from jax.experimental.pallas import tpu as pltpu
from jax.experimental.pallas import tpu_sc as plsc

_NUM_CORES = 2
_NUM_SUBCORES = 16
_NUM_WORKERS = _NUM_CORES * _NUM_SUBCORES


@functools.lru_cache(maxsize=None)
def _make_sc_gather(V, D, B, chunk, group):
  assert B % _NUM_WORKERS == 0
  b_per_w = B // _NUM_WORKERS
  cpg = group // chunk
  n_groups = b_per_w // group
  assert b_per_w % group == 0 and group % chunk == 0

  mesh = plsc.VectorSubcoreMesh(
      core_axis_name="c", subcore_axis_name="s",
      num_cores=_NUM_CORES, num_subcores=_NUM_SUBCORES)

  @functools.partial(
      pl.kernel,
      mesh=mesh,
      out_type=jax.ShapeDtypeStruct((B, D), jnp.float32),
      scratch_types=[
          pltpu.VMEM((b_per_w,), jnp.int32),
          pltpu.VMEM((2, group, D), jnp.float32),
          pltpu.SemaphoreType.DMA,
          pltpu.SemaphoreType.DMA,
      ],
  )
  def gather_kernel(table_hbm, idx_hbm, out_hbm, idx_v, rows_v, gsem, wsem):
    wid = lax.axis_index("s") * _NUM_CORES + lax.axis_index("c")
    base = wid * b_per_w
    pltpu.sync_copy(idx_hbm.at[pl.ds(base, b_per_w)], idx_v)

    def start_group(g):
      buf = rows_v.at[g % 2]
      return [
          pltpu.async_copy(
              table_hbm.at[idx_v.at[pl.ds(g * group + c * chunk, chunk)]],
              buf.at[pl.ds(c * chunk, chunk)], gsem)
          for c in range(cpg)
      ]

    gathers = start_group(0)
    writes = [None] * n_groups
    for g in range(n_groups):
      if g + 1 < n_groups:
        if g >= 1:
          writes[g - 1].wait()
        nxt = start_group(g + 1)
      for cp in gathers:
        cp.wait()
      writes[g] = pltpu.async_copy(
          rows_v.at[g % 2], out_hbm.at[pl.ds(base + g * group, group)], wsem)
      if g + 1 < n_groups:
        gathers = nxt
    writes[n_groups - 2].wait()
    writes[n_groups - 1].wait()

  return gather_kernel


def kernel(word_inputs, word_seq_lengths, char_inputs, char_seq_lengths,
           char_seq_recover, W):
  B, S = word_inputs.shape
  V, D = W.shape
  s_pad = (S + 7) // 8 * 8
  n_pad = s_pad - S
  pad_cols = (jnp.arange(B, dtype=jnp.int32)[:, None] * n_pad
              + jnp.arange(n_pad, dtype=jnp.int32)[None, :]) % V
  idx = jnp.concatenate(
      [word_inputs.astype(jnp.int32), pad_cols], axis=1).reshape(B * s_pad)
  out = _make_sc_gather(V, D, B * s_pad, 112, 224)(W, idx)
  return out.reshape(B, s_pad, D)[:, :S, :]


# speedup vs baseline: 5.6080x; 1.1395x over previous
import functools

import jax
import jax.numpy as jnp
from jax import lax
from jax.experimental import pallas as pl
from jax.experimental.pallas import tpu as pltpu
from jax.experimental.pallas import tpu_sc as plsc

_NUM_CORES = 2
_NUM_SUBCORES = 16
_NUM_WORKERS = _NUM_CORES * _NUM_SUBCORES


@functools.lru_cache(maxsize=None)
def _make_sc_gather(V, D, B, chunk, group):
  assert B % _NUM_WORKERS == 0
  b_per_w = B // _NUM_WORKERS
  cpg = group // chunk
  n_groups = b_per_w // group
  assert b_per_w % group == 0 and group % chunk == 0

  mesh = plsc.VectorSubcoreMesh(
      core_axis_name="c", subcore_axis_name="s",
      num_cores=_NUM_CORES, num_subcores=_NUM_SUBCORES)

  @functools.partial(
      pl.kernel,
      mesh=mesh,
      out_type=jax.ShapeDtypeStruct((B, D), jnp.float32),
      scratch_types=[
          pltpu.VMEM((b_per_w,), jnp.int32),
          pltpu.VMEM((2, group, D), jnp.float32),
          pltpu.SemaphoreType.DMA,
          pltpu.SemaphoreType.DMA,
      ],
  )
  def gather_kernel(table_hbm, idx_hbm, out_hbm, idx_v, rows_v, gsem, wsem):
    wid = lax.axis_index("s") * _NUM_CORES + lax.axis_index("c")
    base = wid * b_per_w
    pltpu.sync_copy(idx_hbm.at[pl.ds(base, b_per_w)], idx_v)

    def start_group(g):
      buf = rows_v.at[g % 2]
      return [
          pltpu.async_copy(
              table_hbm.at[idx_v.at[pl.ds(g * group + c * chunk, chunk)]],
              buf.at[pl.ds(c * chunk, chunk)], gsem)
          for c in range(cpg)
      ]

    gathers = start_group(0)
    writes = [None] * n_groups
    for g in range(n_groups):
      if g + 1 < n_groups:
        if g >= 1:
          writes[g - 1].wait()
        nxt = start_group(g + 1)
      for cp in gathers:
        cp.wait()
      writes[g] = pltpu.async_copy(
          rows_v.at[g % 2], out_hbm.at[pl.ds(base + g * group, group)], wsem)
      if g + 1 < n_groups:
        gathers = nxt
    writes[n_groups - 2].wait()
    writes[n_groups - 1].wait()

  return gather_kernel


@functools.lru_cache(maxsize=None)
def _make_sc_gather_tiled(V, D, B, S, s_pad):
  spw = B // _NUM_WORKERS
  gsz = 8
  n_groups = spw // gsz
  assert n_groups % 2 == 0

  mesh = plsc.VectorSubcoreMesh(
      core_axis_name="c", subcore_axis_name="s",
      num_cores=_NUM_CORES, num_subcores=_NUM_SUBCORES)

  @functools.partial(
      pl.kernel,
      mesh=mesh,
      out_type=jax.ShapeDtypeStruct((B, S, D), jnp.float32),
      compiler_params=pltpu.CompilerParams(use_tc_tiling_on_sc=True),
      scratch_types=[
          pltpu.VMEM((spw * s_pad,), jnp.int32),
          pltpu.VMEM((gsz, S, D), jnp.float32),
          pltpu.VMEM((gsz, S, D), jnp.float32),
          pltpu.SemaphoreType.DMA,
          pltpu.SemaphoreType.DMA,
      ],
  )
  def gather_kernel(table_hbm, idx_hbm, out_hbm, idx_v, buf_a, buf_b, gsem,
                    wsem):
    wid = lax.axis_index("s") * _NUM_CORES + lax.axis_index("c")
    pltpu.sync_copy(idx_hbm.at[pl.ds(wid * spw * s_pad, spw * s_pad)], idx_v)

    def start_group(g, buf):
      return [
          pltpu.async_copy(
              table_hbm.at[idx_v.at[pl.ds((g * gsz + i) * s_pad, S)]],
              buf.at[i], gsem)
          for i in range(gsz)
      ]

    def write_group(g, buf):
      return pltpu.async_copy(
          out_hbm.at[pl.ds(wid * spw + g * gsz, gsz)], buf, wsem)

    def body(t, carry):
      ga = start_group(2 * t, buf_a)
      gb = start_group(2 * t + 1, buf_b)
      for cp in ga:
        cp.wait()
      wa = pltpu.async_copy(buf_a,
                            out_hbm.at[pl.ds(wid * spw + 2 * t * gsz, gsz)],
                            wsem)
      for cp in gb:
        cp.wait()
      wb = pltpu.async_copy(buf_b,
                            out_hbm.at[pl.ds(wid * spw + (2 * t + 1) * gsz,
                                             gsz)],
                            wsem)
      wa.wait()
      wb.wait()
      return carry

    lax.fori_loop(0, n_groups // 2, body, 0)

  return gather_kernel


def kernel(word_inputs, word_seq_lengths, char_inputs, char_seq_lengths,
           char_seq_recover, W):
  B, S = word_inputs.shape
  V, D = W.shape
  s_pad = (S + 7) // 8 * 8
  n_pad = s_pad - S
  pad_cols = (jnp.arange(B, dtype=jnp.int32)[:, None] * n_pad
              + jnp.arange(n_pad, dtype=jnp.int32)[None, :]) % V
  idx = jnp.concatenate(
      [word_inputs.astype(jnp.int32), pad_cols], axis=1).reshape(B * s_pad)
  return _make_sc_gather_tiled(V, D, B, S, s_pad)(W, idx)


# speedup vs baseline: 8.7399x; 1.5585x over previous
import functools

import jax
import jax.numpy as jnp
from jax import lax
from jax.experimental import pallas as pl
from jax.experimental.pallas import tpu as pltpu
from jax.experimental.pallas import tpu_sc as plsc

_NUM_CORES = 2
_NUM_SUBCORES = 16
_NUM_WORKERS = _NUM_CORES * _NUM_SUBCORES


@functools.lru_cache(maxsize=None)
def _make_sc_gather(V, D, B, chunk, group):
  assert B % _NUM_WORKERS == 0
  b_per_w = B // _NUM_WORKERS
  cpg = group // chunk
  n_groups = b_per_w // group
  assert b_per_w % group == 0 and group % chunk == 0

  mesh = plsc.VectorSubcoreMesh(
      core_axis_name="c", subcore_axis_name="s",
      num_cores=_NUM_CORES, num_subcores=_NUM_SUBCORES)

  @functools.partial(
      pl.kernel,
      mesh=mesh,
      out_type=jax.ShapeDtypeStruct((B, D), jnp.float32),
      scratch_types=[
          pltpu.VMEM((b_per_w,), jnp.int32),
          pltpu.VMEM((2, group, D), jnp.float32),
          pltpu.SemaphoreType.DMA,
          pltpu.SemaphoreType.DMA,
      ],
  )
  def gather_kernel(table_hbm, idx_hbm, out_hbm, idx_v, rows_v, gsem, wsem):
    wid = lax.axis_index("s") * _NUM_CORES + lax.axis_index("c")
    base = wid * b_per_w
    pltpu.sync_copy(idx_hbm.at[pl.ds(base, b_per_w)], idx_v)

    def start_group(g):
      buf = rows_v.at[g % 2]
      return [
          pltpu.async_copy(
              table_hbm.at[idx_v.at[pl.ds(g * group + c * chunk, chunk)]],
              buf.at[pl.ds(c * chunk, chunk)], gsem)
          for c in range(cpg)
      ]

    gathers = start_group(0)
    writes = [None] * n_groups
    for g in range(n_groups):
      if g + 1 < n_groups:
        if g >= 1:
          writes[g - 1].wait()
        nxt = start_group(g + 1)
      for cp in gathers:
        cp.wait()
      writes[g] = pltpu.async_copy(
          rows_v.at[g % 2], out_hbm.at[pl.ds(base + g * group, group)], wsem)
      if g + 1 < n_groups:
        gathers = nxt
    writes[n_groups - 2].wait()
    writes[n_groups - 1].wait()

  return gather_kernel


@functools.lru_cache(maxsize=None)
def _make_sc_gather_tiled(V, D, B, S, s_pad):
  spw = B // _NUM_WORKERS
  gsz = 8
  n_groups = spw // gsz
  assert n_groups % 2 == 0

  mesh = plsc.VectorSubcoreMesh(
      core_axis_name="c", subcore_axis_name="s",
      num_cores=_NUM_CORES, num_subcores=_NUM_SUBCORES)

  @functools.partial(
      pl.kernel,
      mesh=mesh,
      out_type=jax.ShapeDtypeStruct((B, S, D), jnp.float32),
      compiler_params=pltpu.CompilerParams(use_tc_tiling_on_sc=True),
      scratch_types=[
          pltpu.VMEM((spw * s_pad,), jnp.int32),
          pltpu.VMEM((gsz, S, D), jnp.float32),
          pltpu.VMEM((gsz, S, D), jnp.float32),
          pltpu.SemaphoreType.DMA,
          pltpu.SemaphoreType.DMA,
      ],
  )
  def gather_kernel(table_hbm, idx_hbm, out_hbm, idx_v, buf_a, buf_b, gsem,
                    wsem):
    wid = lax.axis_index("s") * _NUM_CORES + lax.axis_index("c")
    pltpu.sync_copy(idx_hbm.at[pl.ds(wid * spw * s_pad, spw * s_pad)], idx_v)

    def start_group(g, buf):
      return [
          pltpu.async_copy(
              table_hbm.at[idx_v.at[pl.ds((g * gsz + i) * s_pad, S)]],
              buf.at[i], gsem)
          for i in range(gsz)
      ]

    def write_group(g, buf):
      return pltpu.async_copy(
          out_hbm.at[pl.ds(wid * spw + g * gsz, gsz)], buf, wsem)

    def body(t, carry):
      ga = start_group(2 * t, buf_a)
      gb = start_group(2 * t + 1, buf_b)
      for cp in ga:
        cp.wait()
      wa = pltpu.async_copy(buf_a,
                            out_hbm.at[pl.ds(wid * spw + 2 * t * gsz, gsz)],
                            wsem)
      for cp in gb:
        cp.wait()
      wb = pltpu.async_copy(buf_b,
                            out_hbm.at[pl.ds(wid * spw + (2 * t + 1) * gsz,
                                             gsz)],
                            wsem)
      wa.wait()
      wb.wait()
      return carry

    lax.fori_loop(0, n_groups // 2, body, 0)

  return gather_kernel


def kernel(word_inputs, word_seq_lengths, char_inputs, char_seq_lengths,
           char_seq_recover, W):
  B, S = word_inputs.shape
  V, D = W.shape
  idx = word_inputs.astype(jnp.int32).T.reshape(B * S)
  out = _make_sc_gather(V, D, B * S, 80, 400)(W, idx)
  return out.reshape(S, B, D).transpose(1, 0, 2)
